# baseline (speedup 1.0000x reference)
import functools

import jax
import jax.numpy as jnp
from jax import lax
from jax.experimental import pallas as pl
from jax.experimental.pallas import tpu as pltpu
from jax.experimental.pallas import tpu_sc as plsc

F32 = jnp.float32
I32 = jnp.int32

N = 10000
D = 256
E = 160000
NP = 10240
NW = 32
RPT = NP // NW
EPAD = 163840
EPT = EPAD // NW
K = 128
GRP = 8
LIST = K + 16 * GRP + 16
DUMP = K + 16 * GRP
C = 2048
NCH = EPAD // C
GPC = C // 16
HROWS = NP // 16 + 16


def _score_call(x, p2d, nrm2d):
    def body(x_ref, p_ref, n_ref, o_ref):
        xb = x_ref[...].astype(jnp.bfloat16)
        pb = p_ref[...].astype(jnp.bfloat16)
        dn = (((1,), (0,)), ((), ()))
        s = lax.dot_general(xb, pb, dn, preferred_element_type=F32)
        o_ref[...] = jnp.tanh(s[:, 0:1] / n_ref[0, 0])

    return pl.pallas_call(
        body, out_shape=jax.ShapeDtypeStruct((N, 1), F32))(x, p2d, nrm2d)


def _mesh():
    return plsc.VectorSubcoreMesh(core_axis_name="c", subcore_axis_name="s")


def _sc_params():
    return pltpu.CompilerParams(needs_layout_passes=False)


def _deg_call(dst2):
    @functools.partial(
        pl.kernel,
        out_type=jax.ShapeDtypeStruct((NW, NP // 16, 16), F32),
        mesh=_mesh(),
        scratch_types=[
            pltpu.VMEM((EPT,), I32),
            pltpu.VMEM((HROWS, 16), F32),
        ],
        compiler_params=_sc_params(),
    )
    def deg_kernel(dst_hbm, degs_hbm, dstv, hist):
        c = lax.axis_index("c")
        s = lax.axis_index("s")
        wid = c * 16 + s
        lane = lax.iota(I32, 16)
        pltpu.sync_copy(dst_hbm.at[wid], dstv)

        def zero_body(r, _):
            hist[r, pl.ds(0, 16)] = jnp.zeros((16,), F32)
            return 0

        lax.fori_loop(0, HROWS, zero_body, 0)

        def add_one(dk):
            vec = (lane == (dk & 15)).astype(F32)
            plsc.addupdate(hist.at[dk >> 4, pl.ds(0, 16)], vec)

        def grp(g, _):
            rv = dstv[pl.ds(g * 16, 16)]
            for k in range(16):
                add_one(rv[k])
            return 0

        lax.fori_loop(0, EPT // 16, grp, 0)

        pltpu.sync_copy(hist.at[pl.ds(0, NP // 16)], degs_hbm.at[wid])

    return _call_deg(deg_kernel, dst2)


def _call_deg(k, dst2):
    return k(dst2)


def _topk_gru_xwp_call(x_pad, s80, w_ih, w_hh, b_ih, b_hh, w0, degs):
    blk = 1024

    def body(xf_ref, s_ref, wih_ref, whh_ref, bih_ref, bhh_ref, w0_ref,
             xb_ref, dg_ref, o_ref, xt_ref, w_ref):
        i = pl.program_id(0)

        @pl.when(i == 0)
        def _():
            rows = lax.broadcasted_iota(I32, (NP // 128, 128), 0)
            cols = lax.broadcasted_iota(I32, (NP // 128, 128), 1)
            idxm = rows * 128 + cols

            def step(k, sc):
                m = jnp.max(sc)
                first = jnp.min(jnp.where(sc == m, idxm,
                                          jnp.int32(1 << 30)))
                row = xf_ref[pl.ds(first, 1), :]
                xt_ref[pl.ds(k, 1), :] = row * m
                return jnp.where(idxm == first, -jnp.inf, sc)

            lax.fori_loop(0, D, step, s_ref[...])

            xt = xt_ref[...]
            w0 = w0_ref[...]
            dn = (((1,), (1,)), ((), ()))
            gi = lax.dot_general(xt, wih_ref[...], dn,
                                 preferred_element_type=F32) + bih_ref[...]
            gh = lax.dot_general(w0, whh_ref[...], dn,
                                 preferred_element_type=F32) + bhh_ref[...]
            r = jax.nn.sigmoid(gi[:, :D] + gh[:, :D])
            z = jax.nn.sigmoid(gi[:, D:2 * D] + gh[:, D:2 * D])
            n = jnp.tanh(gi[:, 2 * D:] + r * gh[:, 2 * D:])
            w_ref[...] = (1.0 - z) * n + z * w0

        deg = jnp.sum(dg_ref[...], axis=0, keepdims=True) + 1.0
        dinv = lax.rsqrt(jnp.transpose(deg, (1, 0)))
        xw = jnp.dot(xb_ref[...], w_ref[...], preferred_element_type=F32)
        o_ref[...] = xw * dinv

    return pl.pallas_call(
        body,
        grid=(NP // blk,),
        in_specs=[
            pl.BlockSpec((NP, D), lambda i: (0, 0)),
            pl.BlockSpec((NP // 128, 128), lambda i: (0, 0)),
            pl.BlockSpec((3 * D, D), lambda i: (0, 0)),
            pl.BlockSpec((3 * D, D), lambda i: (0, 0)),
            pl.BlockSpec((1, 3 * D), lambda i: (0, 0)),
            pl.BlockSpec((1, 3 * D), lambda i: (0, 0)),
            pl.BlockSpec((D, D), lambda i: (0, 0)),
            pl.BlockSpec((blk, D), lambda i: (i, 0)),
            pl.BlockSpec((NW, blk), lambda i: (0, i)),
        ],
        out_specs=pl.BlockSpec((blk, D), lambda i: (i, 0)),
        out_shape=jax.ShapeDtypeStruct((NP, D), F32),
        scratch_shapes=[pltpu.VMEM((D, D), F32), pltpu.VMEM((D, D), F32)],
    )(x_pad, s80, w_ih, w_hh, b_ih, b_hh, w0, x_pad, degs)


def _final_call(hpre, degs, lin_w, lin_b2d):
    blk = 1024

    def body(h_ref, dg_ref, w_ref, b_ref, o_ref):
        deg = jnp.sum(dg_ref[...], axis=0, keepdims=True) + 1.0
        dinv = lax.rsqrt(jnp.transpose(deg, (1, 0)))
        h = jnp.maximum(h_ref[...] * dinv, 0.0)
        dn = (((1,), (1,)), ((), ()))
        o_ref[...] = lax.dot_general(
            h, w_ref[...], dn, preferred_element_type=F32) + b_ref[...]

    return pl.pallas_call(
        body,
        grid=(NP // blk,),
        in_specs=[
            pl.BlockSpec((blk, D), lambda i: (i, 0)),
            pl.BlockSpec((NW, blk), lambda i: (0, i)),
            pl.BlockSpec((D, D), lambda i: (0, 0)),
            pl.BlockSpec((1, D), lambda i: (0, 0)),
        ],
        out_specs=pl.BlockSpec((blk, D), lambda i: (i, 0)),
        out_shape=jax.ShapeDtypeStruct((NP, D), F32),
    )(hpre, degs, lin_w, lin_b2d)


def _edge_call(xwp, src2, dst2):
    @functools.partial(
        pl.kernel,
        out_type=jax.ShapeDtypeStruct((NP, D), F32),
        mesh=_mesh(),
        scratch_types=[
            pltpu.VMEM((RPT, D), F32),
            pltpu.VMEM((K, D), F32),
            pltpu.VMEM((C,), I32),
            pltpu.VMEM((C,), I32),
            pltpu.VMEM((C,), I32),
            pltpu.VMEM((C,), I32),
            pltpu.VMEM((LIST,), I32),
            pltpu.VMEM((LIST,), I32),
            pltpu.VMEM((K,), I32),
            pltpu.VMEM((K,), I32),
            pltpu.SemaphoreType.DMA,
            pltpu.SemaphoreType.DMA,
            pltpu.SemaphoreType.DMA,
        ],
        compiler_params=_sc_params(),
    )
    def edge_kernel(xwp_hbm, src_hbm, dst_hbm, hpre_hbm,
                    h, rows, src_a, dst_a, src_b, dst_b,
                    slist, dlist, sbat, dbat, sem_a, sem_b, gsem):
        c = lax.axis_index("c")
        s = lax.axis_index("s")
        wid = c * 16 + s
        lo = wid * RPT
        lane = lax.iota(I32, 16)

        pltpu.sync_copy(xwp_hbm.at[pl.ds(lo, RPT)], h)

        def start_chunk(ch, srcb, dstb, semb):
            pltpu.async_copy(src_hbm.at[ch], srcb, semb)
            pltpu.async_copy(dst_hbm.at[ch], dstb, semb)

        def wait_chunk(ch, srcb, dstb, semb):
            pltpu.make_async_copy(src_hbm.at[ch], srcb, semb).wait()
            pltpu.make_async_copy(dst_hbm.at[ch], dstb, semb).wait()

        def accumulate():
            def acc_grp(gg, _):
                rv = dbat[pl.ds(gg * 16, 16)]
                for k in range(16):
                    r = rv[k]
                    j = gg * 16 + k
                    for i in range(D // 16):
                        plsc.addupdate(h.at[r, pl.ds(i * 16, 16)],
                                       rows[j, pl.ds(i * 16, 16)])
                return 0

            lax.fori_loop(0, K // 16, acc_grp, 0)

        def snapshot_and_fire():
            for i in range(K // 16):
                sbat[pl.ds(i * 16, 16)] = slist[pl.ds(i * 16, 16)]
                dbat[pl.ds(i * 16, 16)] = dlist[pl.ds(i * 16, 16)]
            pltpu.async_copy(xwp_hbm.at[sbat], rows, gsem)

        def wait_gather():
            pltpu.make_async_copy(xwp_hbm.at[sbat], rows, gsem).wait()

        def scan_chunk(srcb, dstb, carry):
            def block(b, cp):
                cur, pending = cp
                for gg in range(GRP):
                    off = b * GRP * 16 + gg * 16
                    d = dstb[pl.ds(off, 16)]
                    sv = srcb[pl.ds(off, 16)]
                    own = (d >= lo) & (d < lo + RPT)
                    inc = plsc.cumsum(own.astype(I32))
                    pos = jnp.where(own, cur + inc - 1, DUMP + lane)
                    plsc.store_scatter(slist, [pos], sv)
                    plsc.store_scatter(dlist, [pos], d - lo)
                    cur = cur + inc[15]
                full = cur >= K

                @pl.when(jnp.logical_and(full, pending == 1))
                def _():
                    wait_gather()
                    accumulate()

                @pl.when(full)
                def _():
                    snapshot_and_fire()
                    for i in range(GRP):
                        v1 = slist[pl.ds(K + i * 16, 16)]
                        v2 = dlist[pl.ds(K + i * 16, 16)]
                        slist[pl.ds(i * 16, 16)] = v1
                        dlist[pl.ds(i * 16, 16)] = v2

                cur = jnp.where(full, cur - K, cur)
                pending = jnp.where(full, 1, pending)
                return cur, pending

            return lax.fori_loop(0, GPC // GRP, block, carry)

        start_chunk(0, src_a, dst_a, sem_a)

        def chunk_pair(ci, carry):
            start_chunk(2 * ci + 1, src_b, dst_b, sem_b)
            wait_chunk(2 * ci, src_a, dst_a, sem_a)
            carry = scan_chunk(src_a, dst_a, carry)

            @pl.when(ci < NCH // 2 - 1)
            def _():
                start_chunk(2 * ci + 2, src_a, dst_a, sem_a)

            wait_chunk(2 * ci + 1, src_b, dst_b, sem_b)
            return scan_chunk(src_b, dst_b, carry)

        cur, pending = lax.fori_loop(0, NCH // 2, chunk_pair, (0, 0))

        @pl.when(pending == 1)
        def _():
            wait_gather()
            accumulate()

        for i in range(K // 16):
            padsrc = N + i * 16 + lane
            m = (i * 16 + lane) < cur
            v = slist[pl.ds(i * 16, 16)]
            vd = dlist[pl.ds(i * 16, 16)]
            slist[pl.ds(i * 16, 16)] = jnp.where(m, v, padsrc)
            dlist[pl.ds(i * 16, 16)] = jnp.where(m, vd, 0)
        snapshot_and_fire()
        wait_gather()
        accumulate()

        pltpu.sync_copy(h, hpre_hbm.at[pl.ds(lo, RPT)])

    return edge_kernel(xwp, src2, dst2)


def kernel(x, edge_index, pool_p, gru_W_ih, gru_W_hh, gru_b_ih, gru_b_hh,
           W0, lin_W, lin_b):
    x_pad = jnp.pad(x, ((0, NP - N), (0, 0)))
    p2d = pool_p.reshape(1, D)

    nrm2d = jnp.linalg.norm(pool_p).reshape(1, 1)
    pmat = jnp.zeros((D, 128), F32).at[:, 0].set(pool_p)
    s = _score_call(x, pmat, nrm2d)
    s80 = jnp.pad(s, ((0, NP - N), (0, 0)),
                  constant_values=-1e30).reshape(NP // 128, 128)
    srcp = jnp.concatenate(
        [edge_index[0], jnp.full((EPAD - E,), N, I32)])
    dstp = jnp.concatenate(
        [edge_index[1], jnp.full((EPAD - E,), NP, I32)])
    degs = _deg_call(dstp.reshape(NW, EPT)).reshape(NW, NP)

    xwp = _topk_gru_xwp_call(x_pad, s80, gru_W_ih, gru_W_hh,
                             gru_b_ih.reshape(1, 3 * D),
                             gru_b_hh.reshape(1, 3 * D), W0, degs)

    src_chunks = srcp.reshape(NCH, C)
    dst_chunks = dstp.reshape(NCH, C)
    hpre = _edge_call(xwp, src_chunks, dst_chunks)

    out = _final_call(hpre, degs, lin_W, lin_b.reshape(1, D))
    return out[:N]


# speedup vs baseline: 1.0095x; 1.0095x over previous
import functools

import jax
import jax.numpy as jnp
from jax import lax
from jax.experimental import pallas as pl
from jax.experimental.pallas import tpu as pltpu
from jax.experimental.pallas import tpu_sc as plsc

F32 = jnp.float32
I32 = jnp.int32

N = 10000
D = 256
E = 160000
NP = 10240
NW = 32
RPT = NP // NW
EPAD = 163840
EPT = EPAD // NW
K = 128
GRP = 8
LIST = K + 16 * GRP + 16
DUMP = K + 16 * GRP
C = 2048
NCH = EPAD // C
GPC = C // 16
HROWS = NP // 16 + 16


def _score_call(x, p2d, nrm2d):
    def body(x_ref, p_ref, n_ref, o_ref):
        xb = x_ref[...].astype(jnp.bfloat16)
        pb = p_ref[...].astype(jnp.bfloat16)
        dn = (((1,), (0,)), ((), ()))
        s = lax.dot_general(xb, pb, dn, preferred_element_type=F32)
        o_ref[...] = jnp.tanh(s[:, 0:1] / n_ref[0, 0])

    return pl.pallas_call(
        body, out_shape=jax.ShapeDtypeStruct((N, 1), F32))(x, p2d, nrm2d)


def _mesh():
    return plsc.VectorSubcoreMesh(core_axis_name="c", subcore_axis_name="s")


def _sc_params():
    return pltpu.CompilerParams(needs_layout_passes=False)


def _deg_call(dst2):
    @functools.partial(
        pl.kernel,
        out_type=jax.ShapeDtypeStruct((NW, NP // 16, 16), F32),
        mesh=_mesh(),
        scratch_types=[
            pltpu.VMEM((EPT,), I32),
            pltpu.VMEM((HROWS, 16), F32),
        ],
        compiler_params=_sc_params(),
    )
    def deg_kernel(dst_hbm, degs_hbm, dstv, hist):
        c = lax.axis_index("c")
        s = lax.axis_index("s")
        wid = c * 16 + s
        lane = lax.iota(I32, 16)
        pltpu.sync_copy(dst_hbm.at[wid], dstv)

        def zero_body(r, _):
            hist[r, pl.ds(0, 16)] = jnp.zeros((16,), F32)
            return 0

        lax.fori_loop(0, HROWS, zero_body, 0)

        def add_one(dk):
            vec = (lane == (dk & 15)).astype(F32)
            plsc.addupdate(hist.at[dk >> 4, pl.ds(0, 16)], vec)

        def grp(g, _):
            rv = dstv[pl.ds(g * 16, 16)]
            for k in range(16):
                add_one(rv[k])
            return 0

        lax.fori_loop(0, EPT // 16, grp, 0)

        pltpu.sync_copy(hist.at[pl.ds(0, NP // 16)], degs_hbm.at[wid])

    return _call_deg(deg_kernel, dst2)


def _call_deg(k, dst2):
    return k(dst2)


def _topk_gru_xwp_call(x_pad, s80, w_ih, w_hh, b_ih, b_hh, w0, degs):
    blk = 1024

    def body(xf_ref, s_ref, wih_ref, whh_ref, bih_ref, bhh_ref, w0_ref,
             xb_ref, dg_ref, o_ref, xt_ref, w_ref):
        i = pl.program_id(0)

        @pl.when(i == 0)
        def _():
            rows = lax.broadcasted_iota(I32, (NP // 128, 128), 0)
            cols = lax.broadcasted_iota(I32, (NP // 128, 128), 1)
            idxm = rows * 128 + cols

            def step(k, sc):
                m = jnp.max(sc)
                first = jnp.min(jnp.where(sc == m, idxm,
                                          jnp.int32(1 << 30)))
                row = xf_ref[pl.ds(first, 1), :]
                xt_ref[pl.ds(k, 1), :] = row * m
                return jnp.where(idxm == first, -jnp.inf, sc)

            lax.fori_loop(0, D, step, s_ref[...])

            xt = xt_ref[...]
            w0 = w0_ref[...]
            dn = (((1,), (1,)), ((), ()))
            gi = lax.dot_general(xt, wih_ref[...], dn,
                                 preferred_element_type=F32) + bih_ref[...]
            gh = lax.dot_general(w0, whh_ref[...], dn,
                                 preferred_element_type=F32) + bhh_ref[...]
            r = jax.nn.sigmoid(gi[:, :D] + gh[:, :D])
            z = jax.nn.sigmoid(gi[:, D:2 * D] + gh[:, D:2 * D])
            n = jnp.tanh(gi[:, 2 * D:] + r * gh[:, 2 * D:])
            w_ref[...] = (1.0 - z) * n + z * w0

        deg = jnp.sum(dg_ref[...], axis=0, keepdims=True) + 1.0
        dinv = lax.rsqrt(jnp.transpose(deg, (1, 0)))
        xw = jnp.dot(xb_ref[...], w_ref[...], preferred_element_type=F32)
        o_ref[...] = xw * dinv

    return pl.pallas_call(
        body,
        grid=(NP // blk,),
        in_specs=[
            pl.BlockSpec((NP, D), lambda i: (0, 0)),
            pl.BlockSpec((NP // 128, 128), lambda i: (0, 0)),
            pl.BlockSpec((3 * D, D), lambda i: (0, 0)),
            pl.BlockSpec((3 * D, D), lambda i: (0, 0)),
            pl.BlockSpec((1, 3 * D), lambda i: (0, 0)),
            pl.BlockSpec((1, 3 * D), lambda i: (0, 0)),
            pl.BlockSpec((D, D), lambda i: (0, 0)),
            pl.BlockSpec((blk, D), lambda i: (i, 0)),
            pl.BlockSpec((NW, blk), lambda i: (0, i)),
        ],
        out_specs=pl.BlockSpec((blk, D), lambda i: (i, 0)),
        out_shape=jax.ShapeDtypeStruct((NP, D), F32),
        scratch_shapes=[pltpu.VMEM((D, D), F32), pltpu.VMEM((D, D), F32)],
    )(x_pad, s80, w_ih, w_hh, b_ih, b_hh, w0, x_pad, degs)


def _final_call(hpre, degs, lin_w, lin_b2d):
    blk = 1024

    def body(h_ref, dg_ref, w_ref, b_ref, o_ref):
        deg = jnp.sum(dg_ref[...], axis=0, keepdims=True) + 1.0
        dinv = lax.rsqrt(jnp.transpose(deg, (1, 0)))
        h = jnp.maximum(h_ref[...] * dinv, 0.0)
        dn = (((1,), (1,)), ((), ()))
        o_ref[...] = lax.dot_general(
            h, w_ref[...], dn, preferred_element_type=F32) + b_ref[...]

    return pl.pallas_call(
        body,
        grid=(NP // blk,),
        in_specs=[
            pl.BlockSpec((blk, D), lambda i: (i, 0)),
            pl.BlockSpec((NW, blk), lambda i: (0, i)),
            pl.BlockSpec((D, D), lambda i: (0, 0)),
            pl.BlockSpec((1, D), lambda i: (0, 0)),
        ],
        out_specs=pl.BlockSpec((blk, D), lambda i: (i, 0)),
        out_shape=jax.ShapeDtypeStruct((NP, D), F32),
    )(hpre, degs, lin_w, lin_b2d)


def _edge_call(xwp, src2, dst2):
    @functools.partial(
        pl.kernel,
        out_type=jax.ShapeDtypeStruct((NP, D), F32),
        mesh=_mesh(),
        scratch_types=[
            pltpu.VMEM((RPT, D), F32),
            pltpu.VMEM((K, D), F32),
            pltpu.VMEM((C,), I32),
            pltpu.VMEM((C,), I32),
            pltpu.VMEM((C,), I32),
            pltpu.VMEM((C,), I32),
            pltpu.VMEM((LIST,), I32),
            pltpu.VMEM((K,), I32),
            pltpu.VMEM((K,), I32),
            pltpu.SemaphoreType.DMA,
            pltpu.SemaphoreType.DMA,
            pltpu.SemaphoreType.DMA,
        ],
        compiler_params=_sc_params(),
    )
    def edge_kernel(xwp_hbm, src_hbm, dst_hbm, hpre_hbm,
                    h, rows, src_a, dst_a, src_b, dst_b,
                    slist, sbat, dbat, sem_a, sem_b, gsem):
        c = lax.axis_index("c")
        s = lax.axis_index("s")
        wid = c * 16 + s
        lo = wid * RPT
        lane = lax.iota(I32, 16)
        dumpv = DUMP + lane

        pltpu.sync_copy(xwp_hbm.at[pl.ds(lo, RPT)], h)

        def start_chunk(ch, srcb, dstb, semb):
            pltpu.async_copy(src_hbm.at[ch], srcb, semb)
            pltpu.async_copy(dst_hbm.at[ch], dstb, semb)

        def wait_chunk(ch, srcb, dstb, semb):
            pltpu.make_async_copy(src_hbm.at[ch], srcb, semb).wait()
            pltpu.make_async_copy(dst_hbm.at[ch], dstb, semb).wait()

        def accumulate():
            def acc_grp(gg, _):
                rv = dbat[pl.ds(gg * 16, 16)]
                for k in range(16):
                    r = rv[k]
                    j = gg * 16 + k
                    for i in range(D // 16):
                        plsc.addupdate(h.at[r, pl.ds(i * 16, 16)],
                                       rows[j, pl.ds(i * 16, 16)])
                return 0

            lax.fori_loop(0, K // 16, acc_grp, 0)

        def snapshot_and_fire():
            for i in range(K // 16):
                v = slist[pl.ds(i * 16, 16)]
                sbat[pl.ds(i * 16, 16)] = v & 16383
                dbat[pl.ds(i * 16, 16)] = v >> 14
            pltpu.async_copy(xwp_hbm.at[sbat], rows, gsem)

        def wait_gather():
            pltpu.make_async_copy(xwp_hbm.at[sbat], rows, gsem).wait()

        def scan_chunk(srcb, dstb, carry):
            def block(b, cp):
                cur, pending = cp
                for gg in range(GRP):
                    off = b * GRP * 16 + gg * 16
                    d = dstb[pl.ds(off, 16)]
                    sv = srcb[pl.ds(off, 16)]
                    dl = d - lo
                    own = dl.astype(jnp.uint32) < jnp.uint32(RPT)
                    inc = plsc.cumsum(own.astype(I32))
                    pos = jnp.where(own, cur + inc - 1, dumpv)
                    plsc.store_scatter(slist, [pos], sv | (dl << 14))
                    cur = cur + inc[15]
                full = cur >= K

                @pl.when(jnp.logical_and(full, pending == 1))
                def _():
                    wait_gather()
                    accumulate()

                @pl.when(full)
                def _():
                    snapshot_and_fire()
                    for i in range(GRP):
                        v1 = slist[pl.ds(K + i * 16, 16)]
                        slist[pl.ds(i * 16, 16)] = v1

                cur = jnp.where(full, cur - K, cur)
                pending = jnp.where(full, 1, pending)
                return cur, pending

            return lax.fori_loop(0, GPC // GRP, block, carry)

        start_chunk(0, src_a, dst_a, sem_a)

        def chunk_pair(ci, carry):
            start_chunk(2 * ci + 1, src_b, dst_b, sem_b)
            wait_chunk(2 * ci, src_a, dst_a, sem_a)
            carry = scan_chunk(src_a, dst_a, carry)

            @pl.when(ci < NCH // 2 - 1)
            def _():
                start_chunk(2 * ci + 2, src_a, dst_a, sem_a)

            wait_chunk(2 * ci + 1, src_b, dst_b, sem_b)
            return scan_chunk(src_b, dst_b, carry)

        cur, pending = lax.fori_loop(0, NCH // 2, chunk_pair, (0, 0))

        @pl.when(pending == 1)
        def _():
            wait_gather()
            accumulate()

        for i in range(K // 16):
            padsrc = N + i * 16 + lane
            m = (i * 16 + lane) < cur
            v = slist[pl.ds(i * 16, 16)]
            slist[pl.ds(i * 16, 16)] = jnp.where(m, v, padsrc)
        snapshot_and_fire()
        wait_gather()
        accumulate()

        pltpu.sync_copy(h, hpre_hbm.at[pl.ds(lo, RPT)])

    return edge_kernel(xwp, src2, dst2)


def kernel(x, edge_index, pool_p, gru_W_ih, gru_W_hh, gru_b_ih, gru_b_hh,
           W0, lin_W, lin_b):
    x_pad = jnp.pad(x, ((0, NP - N), (0, 0)))
    p2d = pool_p.reshape(1, D)

    nrm2d = jnp.linalg.norm(pool_p).reshape(1, 1)
    pmat = jnp.zeros((D, 128), F32).at[:, 0].set(pool_p)
    s = _score_call(x, pmat, nrm2d)
    s80 = jnp.pad(s, ((0, NP - N), (0, 0)),
                  constant_values=-1e30).reshape(NP // 128, 128)
    srcp = jnp.concatenate(
        [edge_index[0], jnp.full((EPAD - E,), N, I32)])
    dstp = jnp.concatenate(
        [edge_index[1], jnp.full((EPAD - E,), NP, I32)])
    degs = _deg_call(dstp.reshape(NW, EPT)).reshape(NW, NP)

    xwp = _topk_gru_xwp_call(x_pad, s80, gru_W_ih, gru_W_hh,
                             gru_b_ih.reshape(1, 3 * D),
                             gru_b_hh.reshape(1, 3 * D), W0, degs)

    src_chunks = srcp.reshape(NCH, C)
    dst_chunks = dstp.reshape(NCH, C)
    hpre = _edge_call(xwp, src_chunks, dst_chunks)

    out = _final_call(hpre, degs, lin_W, lin_b.reshape(1, D))
    return out[:N]
